# baseline (speedup 1.0000x reference)
import jax
import jax.numpy as jnp
from jax.experimental import pallas as pl
from jax.experimental.pallas import tpu as pltpu


def _mish(h):
    sp = jnp.maximum(h, 0.0) + jnp.log1p(jnp.exp(-jnp.abs(h)))
    return h * jnp.tanh(sp)


def _se_kernel(x_ref, w1_ref, b1_ref, w2_ref, b2_ref, o_ref):
    xb = x_ref[0]
    inv_hw = 1.0 / (xb.shape[1] * xb.shape[2])
    s1 = jnp.sum(xb, axis=1)
    m = jnp.sum(s1, axis=1, keepdims=True) * inv_hw
    h = jnp.dot(w1_ref[...], m,
                preferred_element_type=jnp.float32) + b1_ref[...]
    h = _mish(h)
    s = jnp.dot(w2_ref[...], h,
                preferred_element_type=jnp.float32) + b2_ref[...]
    s = jax.nn.sigmoid(s)
    o_ref[0] = xb * s[:, :, None]


def kernel(x, W1, b1, W2, b2, *, interpret=False):
    B, C, H, W = x.shape
    HID = W1.shape[0]
    b1c = b1.reshape(HID, 1)
    b2c = b2.reshape(C, 1)
    return pl.pallas_call(
        _se_kernel,
        out_shape=jax.ShapeDtypeStruct((B, C, H, W), x.dtype),
        grid=(B,),
        in_specs=[
            pl.BlockSpec((1, C, H, W), lambda i: (i, 0, 0, 0)),
            pl.BlockSpec((HID, C), lambda i: (0, 0)),
            pl.BlockSpec((HID, 1), lambda i: (0, 0)),
            pl.BlockSpec((C, HID), lambda i: (0, 0)),
            pl.BlockSpec((C, 1), lambda i: (0, 0)),
        ],
        out_specs=pl.BlockSpec((1, C, H, W), lambda i: (i, 0, 0, 0)),
        compiler_params=pltpu.CompilerParams(
            dimension_semantics=("parallel",),
        ),
        name="se_fused",
        interpret=interpret,
    )(x, W1, b1c, W2, b2c)


# speedup vs baseline: 4.2940x; 4.2940x over previous
import jax
import jax.numpy as jnp
from jax.experimental import pallas as pl
from jax.experimental.pallas import tpu as pltpu


def _mish(h):
    sp = jnp.maximum(h, 0.0) + jnp.log1p(jnp.exp(-jnp.abs(h)))
    return h * jnp.tanh(sp)


def _se_kernel(x_ref, w1t_ref, b1_ref, w2t_ref, b2_ref, o_ref):
    xb = x_ref[0]
    inv_hw = 1.0 / (xb.shape[0] * xb.shape[1])
    s = jnp.sum(xb, axis=0)
    m = jnp.sum(s, axis=0, keepdims=True) * inv_hw
    h = jnp.dot(m, w1t_ref[...],
                preferred_element_type=jnp.float32) + b1_ref[...]
    h = _mish(h)
    g = jnp.dot(h, w2t_ref[...],
                preferred_element_type=jnp.float32) + b2_ref[...]
    g = jax.nn.sigmoid(g)
    o_ref[0] = xb * g


def kernel(x, W1, b1, W2, b2, *, interpret=False):
    B, C, H, W = x.shape
    HID = W1.shape[0]
    xt = x.transpose(0, 2, 3, 1)
    out = pl.pallas_call(
        _se_kernel,
        out_shape=jax.ShapeDtypeStruct((B, H, W, C), x.dtype),
        grid=(B,),
        in_specs=[
            pl.BlockSpec((1, H, W, C), lambda i: (i, 0, 0, 0)),
            pl.BlockSpec((C, HID), lambda i: (0, 0)),
            pl.BlockSpec((1, HID), lambda i: (0, 0)),
            pl.BlockSpec((HID, C), lambda i: (0, 0)),
            pl.BlockSpec((1, C), lambda i: (0, 0)),
        ],
        out_specs=pl.BlockSpec((1, H, W, C), lambda i: (i, 0, 0, 0)),
        compiler_params=pltpu.CompilerParams(
            dimension_semantics=("parallel",),
        ),
        name="se_fused",
        interpret=interpret,
    )(xt, W1.T, b1.reshape(1, HID), W2.T, b2.reshape(1, C))
    return out.transpose(0, 3, 1, 2)
